# baseline (speedup 1.0000x reference)
import functools

import jax
import jax.numpy as jnp
from jax import lax
from jax.experimental import pallas as pl
from jax.experimental.pallas import tpu as pltpu
from jax.experimental.pallas import tpu_sc as plsc

N = 10000
E = 320000
D = 128
NC = 2
NS = 16
NW = NC * NS
EPW = E // NW
CHUNK = 80
NPH = 5
PSTEPS = EPW // (CHUNK * NPH)
RPS = 640
RPS_LAST = N - (NS - 1) * RPS


def _sc_body(y_hbm, srcr_hbm, dstr_hbm, part_hbm,
             ibufs, rbuf0, rbuf1, agg, isem0, isem1, gsem0, gsem1,
             ssem0, ssem1):
    cid = lax.axis_index("c")
    sid = lax.axis_index("s")
    wid = cid * NS + sid

    @pl.loop(0, CHUNK)
    def _(r):
        @pl.loop(0, D, step=16)
        def _(c):
            rbuf0[r, pl.ds(c, 16)] = jnp.zeros((16,), jnp.float32)

    start = sid * RPS

    @pl.when(sid < NS - 1)
    def _():
        @pl.loop(0, RPS, step=CHUNK)
        def _(j):
            pltpu.sync_copy(rbuf0, agg.at[pl.ds(start + j, CHUNK)])

    @pl.when(sid == NS - 1)
    def _():
        @pl.loop(0, RPS_LAST, step=CHUNK)
        def _(j):
            pltpu.sync_copy(rbuf0, agg.at[pl.ds(start + j, CHUNK)])

    plsc.subcore_barrier()

    def i_start(ph, b, sem):
        pltpu.async_copy(srcr_hbm.at[wid, ph], ibufs.at[b, 0], sem)
        pltpu.async_copy(dstr_hbm.at[wid, ph], ibufs.at[b, 1], sem)

    def i_wait(b, sem):
        pltpu.make_async_copy(srcr_hbm.at[0, 0], ibufs.at[b, 0], sem).wait()
        pltpu.make_async_copy(dstr_hbm.at[0, 0], ibufs.at[b, 1], sem).wait()

    HC = CHUNK // 2

    def g_start(b, step, rbuf, sem):
        pltpu.async_copy(y_hbm.at[ibufs.at[b, 0, step, pl.ds(0, HC)]],
                         rbuf.at[pl.ds(0, HC)], sem)
        pltpu.async_copy(y_hbm.at[ibufs.at[b, 0, step, pl.ds(HC, HC)]],
                         rbuf.at[pl.ds(HC, HC)], sem)

    def g_wait(rbuf, sem):
        pltpu.make_async_copy(y_hbm.at[ibufs.at[0, 0, 0, pl.ds(0, HC)]],
                              rbuf.at[pl.ds(0, HC)], sem).wait()
        pltpu.make_async_copy(y_hbm.at[ibufs.at[0, 0, 0, pl.ds(0, HC)]],
                              rbuf.at[pl.ds(HC, HC)], sem).wait()

    def s_start(b, step, rbuf, sem):
        pltpu.async_copy(rbuf, agg.at[ibufs.at[b, 1, step]], sem, add=True)

    def s_wait(rbuf, sem):
        pltpu.make_async_copy(rbuf, agg.at[ibufs.at[0, 1, 0]], sem).wait()

    isems = (isem0, isem1)
    i_start(0, 0, isems[0])
    for ph in range(NPH):
        b = ph % 2
        i_wait(b, isems[b])
        if ph + 1 < NPH:
            i_start(ph + 1, 1 - b, isems[1 - b])

        g_start(b, 0, rbuf0, gsem0)
        g_wait(rbuf0, gsem0)
        s_start(b, 0, rbuf0, ssem0)
        g_start(b, 1, rbuf1, gsem1)

        @pl.loop(1, PSTEPS - 1, step=2)
        def _(i):
            g_wait(rbuf1, gsem1)
            s_start(b, i, rbuf1, ssem1)
            s_wait(rbuf0, ssem0)
            g_start(b, i + 1, rbuf0, gsem0)
            g_wait(rbuf0, gsem0)
            s_start(b, i + 1, rbuf0, ssem0)
            s_wait(rbuf1, ssem1)

            @pl.when(i + 2 < PSTEPS)
            def _():
                g_start(b, i + 2, rbuf1, gsem1)

        s_wait(rbuf0, ssem0)

    plsc.subcore_barrier()

    @pl.when(sid < NS - 1)
    def _():
        @pl.loop(0, RPS, step=CHUNK)
        def _(j):
            pltpu.sync_copy(agg.at[pl.ds(start + j, CHUNK)],
                            part_hbm.at[cid, pl.ds(start + j, CHUNK)])

    @pl.when(sid == NS - 1)
    def _():
        @pl.loop(0, RPS_LAST, step=CHUNK)
        def _(j):
            pltpu.sync_copy(agg.at[pl.ds(start + j, CHUNK)],
                            part_hbm.at[cid, pl.ds(start + j, CHUNK)])


def _sc_aggregate(y, src_r, dst_r):
    mesh = plsc.VectorSubcoreMesh(core_axis_name="c", subcore_axis_name="s")
    kfn = pl.kernel(
        _sc_body,
        out_type=jax.ShapeDtypeStruct((NC, N, D), jnp.float32),
        mesh=mesh,
        scratch_types=[
            pltpu.VMEM((2, 2, PSTEPS, CHUNK), jnp.int32),
            pltpu.VMEM((CHUNK, D), jnp.float32),
            pltpu.VMEM((CHUNK, D), jnp.float32),
            pltpu.VMEM_SHARED((N, D), jnp.float32),
            pltpu.SemaphoreType.DMA,
            pltpu.SemaphoreType.DMA,
            pltpu.SemaphoreType.DMA,
            pltpu.SemaphoreType.DMA,
            pltpu.SemaphoreType.DMA,
            pltpu.SemaphoreType.DMA,
        ],
    )
    return kfn(y, src_r, dst_r)


def _relu_body(x_ref, o_ref):
    o_ref[...] = jnp.maximum(x_ref[...], 0.0)


def _relu(x):
    return pl.pallas_call(
        _relu_body,
        out_shape=jax.ShapeDtypeStruct(x.shape, x.dtype),
    )(x)


def _dense_body(eps_ref, x_ref, p_ref, W1_ref, b1_ref, g1_ref, be1_ref,
                W2_ref, b2_ref, go_ref, bo_ref, o_ref, *, final):
    scale = 1.0 + eps_ref[0]
    h = x_ref[...] * scale + p_ref[0] + p_ref[1]
    h = jnp.dot(h, W1_ref[...], preferred_element_type=jnp.float32)
    h = h + b1_ref[...]
    m = jnp.mean(h, axis=0, keepdims=True)
    v = jnp.mean(h * h, axis=0, keepdims=True) - m * m
    h = (h - m) * lax.rsqrt(v + 1e-5) * g1_ref[...] + be1_ref[...]
    h = jnp.maximum(h, 0.0)
    h = jnp.dot(h, W2_ref[...], preferred_element_type=jnp.float32)
    h = h + b2_ref[...]
    m = jnp.mean(h, axis=0, keepdims=True)
    v = jnp.mean(h * h, axis=0, keepdims=True) - m * m
    h = (h - m) * lax.rsqrt(v + 1e-5) * go_ref[...] + bo_ref[...]
    if not final:
        h = jnp.maximum(h, 0.0)
    o_ref[...] = h


def _dense(x, parts, eps, W1, b1, g1, be1, W2, b2, go, bo, final):
    vecs = [v.reshape(1, D) for v in (b1, g1, be1, b2, go, bo)]
    return pl.pallas_call(
        functools.partial(_dense_body, final=final),
        out_shape=jax.ShapeDtypeStruct((N, D), jnp.float32),
        in_specs=[pl.BlockSpec(memory_space=pltpu.SMEM)] +
                 [pl.BlockSpec()] * 10,
    )(eps, x, parts, W1, vecs[0], vecs[1], vecs[2], W2, vecs[3],
      vecs[4], vecs[5])


def kernel(x, edge_index, eps0, W1_0, b1_0, g1_0, be1_0, W2_0, b2_0, go_0,
           bo_0, eps1, W1_1, b1_1, g1_1, be1_1, W2_1, b2_1, go_1, bo_1):
    src_r = edge_index[0].reshape(NW, NPH, PSTEPS, CHUNK)
    dst_r = edge_index[1].reshape(NW, NPH, PSTEPS, CHUNK)

    y0 = _relu(x)
    parts0 = _sc_aggregate(y0, src_r, dst_r)
    h1 = _dense(x, parts0, eps0, W1_0, b1_0, g1_0, be1_0, W2_0, b2_0,
                go_0, bo_0, final=False)
    parts1 = _sc_aggregate(h1, src_r, dst_r)
    out = _dense(h1, parts1, eps1, W1_1, b1_1, g1_1, be1_1, W2_1, b2_1,
                 go_1, bo_1, final=True)
    return out


# speedup vs baseline: 1.0032x; 1.0032x over previous
import functools

import jax
import jax.numpy as jnp
from jax import lax
from jax.experimental import pallas as pl
from jax.experimental.pallas import tpu as pltpu
from jax.experimental.pallas import tpu_sc as plsc

N = 10000
E = 320000
D = 128
NC = 2
NS = 16
NW = NC * NS
EPW = E // NW
CHUNK = 80
NPH = 5
PSTEPS = EPW // (CHUNK * NPH)
RPS = 640
RPS_LAST = N - (NS - 1) * RPS


def _sc_body(y_hbm, srcr_hbm, dstr_hbm, part_hbm,
             ibufs, rbuf0, rbuf1, agg, isem0, isem1, gsem0, gsem1,
             ssem0, ssem1):
    cid = lax.axis_index("c")
    sid = lax.axis_index("s")
    wid = cid * NS + sid

    def i_start(ph, b, sem):
        pltpu.async_copy(srcr_hbm.at[wid, ph], ibufs.at[b, 0], sem)
        pltpu.async_copy(dstr_hbm.at[wid, ph], ibufs.at[b, 1], sem)

    def i_wait(b, sem):
        pltpu.make_async_copy(srcr_hbm.at[0, 0], ibufs.at[b, 0], sem).wait()
        pltpu.make_async_copy(dstr_hbm.at[0, 0], ibufs.at[b, 1], sem).wait()

    i_start(0, 0, isem0)

    @pl.loop(0, CHUNK)
    def _(r):
        @pl.loop(0, D, step=16)
        def _(c):
            rbuf0[r, pl.ds(c, 16)] = jnp.zeros((16,), jnp.float32)

    start = sid * RPS

    def fire_drain(src_of, dst_of, nrows, sem):
        @pl.loop(0, nrows, step=CHUNK)
        def _(j):
            pltpu.async_copy(src_of(j), dst_of(j), sem)

        @pl.loop(0, nrows, step=CHUNK)
        def _(j):
            pltpu.make_async_copy(src_of(j), dst_of(j), sem).wait()

    @pl.when(sid < NS - 1)
    def _():
        fire_drain(lambda j: rbuf0, lambda j: agg.at[pl.ds(start + j, CHUNK)],
                   RPS, ssem0)

    @pl.when(sid == NS - 1)
    def _():
        fire_drain(lambda j: rbuf0, lambda j: agg.at[pl.ds(start + j, CHUNK)],
                   RPS_LAST, ssem0)

    plsc.subcore_barrier()

    def g_start(b, step, rbuf, sem):
        pltpu.async_copy(y_hbm.at[ibufs.at[b, 0, step]], rbuf, sem)

    def g_wait(rbuf, sem):
        pltpu.make_async_copy(y_hbm.at[ibufs.at[0, 0, 0]], rbuf, sem).wait()

    def s_start(b, step, rbuf, sem):
        pltpu.async_copy(rbuf, agg.at[ibufs.at[b, 1, step]], sem, add=True)

    def s_wait(rbuf, sem):
        pltpu.make_async_copy(rbuf, agg.at[ibufs.at[0, 1, 0]], sem).wait()

    isems = (isem0, isem1)
    for ph in range(NPH):
        b = ph % 2
        i_wait(b, isems[b])
        if ph + 1 < NPH:
            i_start(ph + 1, 1 - b, isems[1 - b])

        g_start(b, 0, rbuf0, gsem0)
        g_wait(rbuf0, gsem0)
        s_start(b, 0, rbuf0, ssem0)
        g_start(b, 1, rbuf1, gsem1)

        @pl.loop(1, PSTEPS - 1, step=2)
        def _(i):
            g_wait(rbuf1, gsem1)
            s_start(b, i, rbuf1, ssem1)
            s_wait(rbuf0, ssem0)
            g_start(b, i + 1, rbuf0, gsem0)
            g_wait(rbuf0, gsem0)
            s_start(b, i + 1, rbuf0, ssem0)
            s_wait(rbuf1, ssem1)

            @pl.when(i + 2 < PSTEPS)
            def _():
                g_start(b, i + 2, rbuf1, gsem1)

        s_wait(rbuf0, ssem0)

    plsc.subcore_barrier()

    @pl.when(sid < NS - 1)
    def _():
        fire_drain(lambda j: agg.at[pl.ds(start + j, CHUNK)],
                   lambda j: part_hbm.at[cid, pl.ds(start + j, CHUNK)],
                   RPS, ssem0)

    @pl.when(sid == NS - 1)
    def _():
        fire_drain(lambda j: agg.at[pl.ds(start + j, CHUNK)],
                   lambda j: part_hbm.at[cid, pl.ds(start + j, CHUNK)],
                   RPS_LAST, ssem0)


def _sc_aggregate(y, src_r, dst_r):
    mesh = plsc.VectorSubcoreMesh(core_axis_name="c", subcore_axis_name="s")
    kfn = pl.kernel(
        _sc_body,
        out_type=jax.ShapeDtypeStruct((NC, N, D), jnp.float32),
        mesh=mesh,
        scratch_types=[
            pltpu.VMEM((2, 2, PSTEPS, CHUNK), jnp.int32),
            pltpu.VMEM((CHUNK, D), jnp.float32),
            pltpu.VMEM((CHUNK, D), jnp.float32),
            pltpu.VMEM_SHARED((N, D), jnp.float32),
            pltpu.SemaphoreType.DMA,
            pltpu.SemaphoreType.DMA,
            pltpu.SemaphoreType.DMA,
            pltpu.SemaphoreType.DMA,
            pltpu.SemaphoreType.DMA,
            pltpu.SemaphoreType.DMA,
        ],
    )
    return kfn(y, src_r, dst_r)


def _relu_body(x_ref, o_ref):
    o_ref[...] = jnp.maximum(x_ref[...], 0.0)


def _relu(x):
    return pl.pallas_call(
        _relu_body,
        out_shape=jax.ShapeDtypeStruct(x.shape, x.dtype),
    )(x)


def _dense_body(eps_ref, x_ref, p_ref, W1_ref, b1_ref, g1_ref, be1_ref,
                W2_ref, b2_ref, go_ref, bo_ref, o_ref, *, final):
    scale = 1.0 + eps_ref[0]
    h = x_ref[...] * scale + p_ref[0] + p_ref[1]
    h = jnp.dot(h, W1_ref[...], preferred_element_type=jnp.float32)
    h = h + b1_ref[...]
    m = jnp.mean(h, axis=0, keepdims=True)
    v = jnp.mean(h * h, axis=0, keepdims=True) - m * m
    h = (h - m) * lax.rsqrt(v + 1e-5) * g1_ref[...] + be1_ref[...]
    h = jnp.maximum(h, 0.0)
    h = jnp.dot(h, W2_ref[...], preferred_element_type=jnp.float32)
    h = h + b2_ref[...]
    m = jnp.mean(h, axis=0, keepdims=True)
    v = jnp.mean(h * h, axis=0, keepdims=True) - m * m
    h = (h - m) * lax.rsqrt(v + 1e-5) * go_ref[...] + bo_ref[...]
    if not final:
        h = jnp.maximum(h, 0.0)
    o_ref[...] = h


def _dense(x, parts, eps, W1, b1, g1, be1, W2, b2, go, bo, final):
    vecs = [v.reshape(1, D) for v in (b1, g1, be1, b2, go, bo)]
    return pl.pallas_call(
        functools.partial(_dense_body, final=final),
        out_shape=jax.ShapeDtypeStruct((N, D), jnp.float32),
        in_specs=[pl.BlockSpec(memory_space=pltpu.SMEM)] +
                 [pl.BlockSpec()] * 10,
    )(eps, x, parts, W1, vecs[0], vecs[1], vecs[2], W2, vecs[3],
      vecs[4], vecs[5])


def kernel(x, edge_index, eps0, W1_0, b1_0, g1_0, be1_0, W2_0, b2_0, go_0,
           bo_0, eps1, W1_1, b1_1, g1_1, be1_1, W2_1, b2_1, go_1, bo_1):
    src_r = edge_index[0].reshape(NW, NPH, PSTEPS, CHUNK)
    dst_r = edge_index[1].reshape(NW, NPH, PSTEPS, CHUNK)

    y0 = _relu(x)
    parts0 = _sc_aggregate(y0, src_r, dst_r)
    h1 = _dense(x, parts0, eps0, W1_0, b1_0, g1_0, be1_0, W2_0, b2_0,
                go_0, bo_0, final=False)
    parts1 = _sc_aggregate(h1, src_r, dst_r)
    out = _dense(h1, parts1, eps1, W1_1, b1_1, g1_1, be1_1, W2_1, b2_1,
                 go_1, bo_1, final=True)
    return out


# speedup vs baseline: 1.0123x; 1.0091x over previous
import functools

import jax
import jax.numpy as jnp
from jax import lax
from jax.experimental import pallas as pl
from jax.experimental.pallas import tpu as pltpu
from jax.experimental.pallas import tpu_sc as plsc

N = 10000
E = 320000
D = 128
NC = 2
NS = 16
NW = NC * NS
EPW = E // NW
CHUNK = 80
NPH = 5
PSTEPS = EPW // (CHUNK * NPH)
RPS = 640
RPS_LAST = N - (NS - 1) * RPS


def _sc_body(y_hbm, srcr_hbm, dstr_hbm, part_hbm,
             ibufs, rbuf0, rbuf1, agg, isem0, isem1, gsem0, gsem1,
             ssem0, ssem1):
    cid = lax.axis_index("c")
    sid = lax.axis_index("s")
    wid = cid * NS + sid

    def i_start(ph, b, sem):
        pltpu.async_copy(srcr_hbm.at[wid, ph], ibufs.at[b, 0], sem)
        pltpu.async_copy(dstr_hbm.at[wid, ph], ibufs.at[b, 1], sem)

    def i_wait(b, sem):
        pltpu.make_async_copy(srcr_hbm.at[0, 0], ibufs.at[b, 0], sem).wait()
        pltpu.make_async_copy(dstr_hbm.at[0, 0], ibufs.at[b, 1], sem).wait()

    i_start(0, 0, isem0)

    @pl.loop(0, CHUNK)
    def _(r):
        @pl.loop(0, D, step=16)
        def _(c):
            rbuf0[r, pl.ds(c, 16)] = jnp.zeros((16,), jnp.float32)

    start = sid * RPS

    def fire_drain(src_of, dst_of, nrows, sem):
        @pl.loop(0, nrows, step=CHUNK)
        def _(j):
            pltpu.async_copy(src_of(j), dst_of(j), sem)

        @pl.loop(0, nrows, step=CHUNK)
        def _(j):
            pltpu.make_async_copy(src_of(j), dst_of(j), sem).wait()

    @pl.when(sid < NS - 1)
    def _():
        fire_drain(lambda j: rbuf0, lambda j: agg.at[pl.ds(start + j, CHUNK)],
                   RPS, ssem0)

    @pl.when(sid == NS - 1)
    def _():
        fire_drain(lambda j: rbuf0, lambda j: agg.at[pl.ds(start + j, CHUNK)],
                   RPS_LAST, ssem0)

    plsc.subcore_barrier()

    def g_start(b, step, rbuf, sem):
        pltpu.async_copy(y_hbm.at[ibufs.at[b, 0, step]], rbuf, sem)

    def g_wait(rbuf, sem):
        pltpu.make_async_copy(y_hbm.at[ibufs.at[0, 0, 0]], rbuf, sem).wait()

    def s_start(b, step, rbuf, sem):
        pltpu.async_copy(rbuf, agg.at[ibufs.at[b, 1, step]], sem, add=True)

    def s_wait(rbuf, sem):
        pltpu.make_async_copy(rbuf, agg.at[ibufs.at[0, 1, 0]], sem).wait()

    isems = (isem0, isem1)
    for ph in range(NPH):
        b = ph % 2
        if b == 0:
            A, B, gA, gB, sA, sB = rbuf0, rbuf1, gsem0, gsem1, ssem0, ssem1
        else:
            A, B, gA, gB, sA, sB = rbuf1, rbuf0, gsem1, gsem0, ssem1, ssem0

        if ph == 0:
            i_wait(0, isem0)
            g_start(0, 0, A, gA)

        g_wait(A, gA)
        s_start(b, 0, A, sA)
        if ph > 0:
            s_wait(B, sB)
        g_start(b, 1, B, gB)
        if ph + 1 < NPH:
            i_start(ph + 1, 1 - b, isems[1 - b])

        @pl.loop(1, PSTEPS - 1, step=2)
        def _(i):
            g_wait(B, gB)
            s_start(b, i, B, sB)
            s_wait(A, sA)
            g_start(b, i + 1, A, gA)
            g_wait(A, gA)
            s_start(b, i + 1, A, sA)
            s_wait(B, sB)

            @pl.when(i + 2 < PSTEPS)
            def _():
                g_start(b, i + 2, B, gB)

        if ph + 1 < NPH:
            i_wait(1 - b, isems[1 - b])
            g_start(1 - b, 0, B, gB)
        else:
            s_wait(A, sA)

    plsc.subcore_barrier()

    @pl.when(sid < NS - 1)
    def _():
        fire_drain(lambda j: agg.at[pl.ds(start + j, CHUNK)],
                   lambda j: part_hbm.at[cid, pl.ds(start + j, CHUNK)],
                   RPS, ssem0)

    @pl.when(sid == NS - 1)
    def _():
        fire_drain(lambda j: agg.at[pl.ds(start + j, CHUNK)],
                   lambda j: part_hbm.at[cid, pl.ds(start + j, CHUNK)],
                   RPS_LAST, ssem0)


def _sc_aggregate(y, src_r, dst_r):
    mesh = plsc.VectorSubcoreMesh(core_axis_name="c", subcore_axis_name="s")
    kfn = pl.kernel(
        _sc_body,
        out_type=jax.ShapeDtypeStruct((NC, N, D), jnp.float32),
        mesh=mesh,
        scratch_types=[
            pltpu.VMEM((2, 2, PSTEPS, CHUNK), jnp.int32),
            pltpu.VMEM((CHUNK, D), jnp.float32),
            pltpu.VMEM((CHUNK, D), jnp.float32),
            pltpu.VMEM_SHARED((N, D), jnp.float32),
            pltpu.SemaphoreType.DMA,
            pltpu.SemaphoreType.DMA,
            pltpu.SemaphoreType.DMA,
            pltpu.SemaphoreType.DMA,
            pltpu.SemaphoreType.DMA,
            pltpu.SemaphoreType.DMA,
        ],
    )
    return kfn(y, src_r, dst_r)


def _relu_body(x_ref, o_ref):
    o_ref[...] = jnp.maximum(x_ref[...], 0.0)


def _relu(x):
    return pl.pallas_call(
        _relu_body,
        out_shape=jax.ShapeDtypeStruct(x.shape, x.dtype),
    )(x)


def _dense_body(eps_ref, x_ref, p_ref, W1_ref, b1_ref, g1_ref, be1_ref,
                W2_ref, b2_ref, go_ref, bo_ref, o_ref, *, final):
    scale = 1.0 + eps_ref[0]
    h = x_ref[...] * scale + p_ref[0] + p_ref[1]
    h = jnp.dot(h, W1_ref[...], preferred_element_type=jnp.float32)
    h = h + b1_ref[...]
    m = jnp.mean(h, axis=0, keepdims=True)
    v = jnp.mean(h * h, axis=0, keepdims=True) - m * m
    h = (h - m) * lax.rsqrt(v + 1e-5) * g1_ref[...] + be1_ref[...]
    h = jnp.maximum(h, 0.0)
    h = jnp.dot(h, W2_ref[...], preferred_element_type=jnp.float32)
    h = h + b2_ref[...]
    m = jnp.mean(h, axis=0, keepdims=True)
    v = jnp.mean(h * h, axis=0, keepdims=True) - m * m
    h = (h - m) * lax.rsqrt(v + 1e-5) * go_ref[...] + bo_ref[...]
    if not final:
        h = jnp.maximum(h, 0.0)
    o_ref[...] = h


def _dense(x, parts, eps, W1, b1, g1, be1, W2, b2, go, bo, final):
    vecs = [v.reshape(1, D) for v in (b1, g1, be1, b2, go, bo)]
    return pl.pallas_call(
        functools.partial(_dense_body, final=final),
        out_shape=jax.ShapeDtypeStruct((N, D), jnp.float32),
        in_specs=[pl.BlockSpec(memory_space=pltpu.SMEM)] +
                 [pl.BlockSpec()] * 10,
    )(eps, x, parts, W1, vecs[0], vecs[1], vecs[2], W2, vecs[3],
      vecs[4], vecs[5])


def kernel(x, edge_index, eps0, W1_0, b1_0, g1_0, be1_0, W2_0, b2_0, go_0,
           bo_0, eps1, W1_1, b1_1, g1_1, be1_1, W2_1, b2_1, go_1, bo_1):
    src_r = edge_index[0].reshape(NW, NPH, PSTEPS, CHUNK)
    dst_r = edge_index[1].reshape(NW, NPH, PSTEPS, CHUNK)

    y0 = _relu(x)
    parts0 = _sc_aggregate(y0, src_r, dst_r)
    h1 = _dense(x, parts0, eps0, W1_0, b1_0, g1_0, be1_0, W2_0, b2_0,
                go_0, bo_0, final=False)
    parts1 = _sc_aggregate(h1, src_r, dst_r)
    out = _dense(h1, parts1, eps1, W1_1, b1_1, g1_1, be1_1, W2_1, b2_1,
                 go_1, bo_1, final=True)
    return out
